# Initial kernel scaffold; baseline (speedup 1.0000x reference)
import jax
import jax.numpy as jnp
from jax.experimental import pallas as pl


def kernel(x, temperature, W1, b1, W2, b2, W3, b3):
    raise NotImplementedError("write your pallas kernel here")


# speedup vs baseline: 1448.4978x; 1448.4978x over previous
import jax
import jax.numpy as jnp
from jax import lax
from jax.experimental import pallas as pl

_K = 5
_BM = 512
_TI = 256


def _mlp_body(x_ref, w1_ref, b1_ref, w2_ref, b2_ref, w3_ref, b3_ref, z_ref):
    h = jnp.dot(x_ref[...], w1_ref[...], preferred_element_type=jnp.float32)
    h = jnp.maximum(h + b1_ref[...], 0.0)
    h = jnp.dot(h, w2_ref[...], preferred_element_type=jnp.float32)
    h = jnp.maximum(h + b2_ref[...], 0.0)
    h = jnp.dot(h, w3_ref[...], preferred_element_type=jnp.float32)
    z_ref[...] = h + b3_ref[...]


def _knn_body(zq_ref, zk_ref, sqc_ref, sqr_ref, vals_ref, idx_ref):
    zq = zq_ref[0]
    zk = zk_ref[0]
    sqq = sqc_ref[0]
    sqk = sqr_ref[0]
    n = zk.shape[0]
    cross = lax.dot_general(zq, zk, (((1,), (1,)), ((), ())),
                            preferred_element_type=jnp.float32)
    dist = jnp.maximum(sqq + sqk - 2.0 * cross, 0.0)
    iota = lax.broadcasted_iota(jnp.int32, dist.shape, 1)
    vals, idxs = [], []
    cur = dist
    for _ in range(_K):
        m = jnp.min(cur, axis=1, keepdims=True)
        im = jnp.min(jnp.where(cur == m, iota, n), axis=1, keepdims=True)
        vals.append(m)
        idxs.append(im)
        cur = jnp.where(iota == im, jnp.inf, cur)
    vals_ref[0] = jnp.concatenate(vals, axis=1)
    idx_ref[0] = jnp.concatenate(idxs, axis=1)


def kernel(x, temperature, W1, b1, W2, b2, W3, b3):
    b, n, d = x.shape
    k = _K
    x2 = x.reshape(b * n, d)
    wspec = pl.BlockSpec((d, d), lambda i: (0, 0))
    bspec = pl.BlockSpec((1, d), lambda i: (0, 0))
    z2 = pl.pallas_call(
        _mlp_body,
        grid=(b * n // _BM,),
        in_specs=[pl.BlockSpec((_BM, d), lambda i: (i, 0)),
                  wspec, bspec, wspec, bspec, wspec, bspec],
        out_specs=pl.BlockSpec((_BM, d), lambda i: (i, 0)),
        out_shape=jax.ShapeDtypeStruct((b * n, d), jnp.float32),
    )(x2, W1.T, b1.reshape(1, d), W2.T, b2.reshape(1, d), W3.T,
      b3.reshape(1, d))
    z = z2.reshape(b, n, d)
    sq = jnp.sum(z * z, axis=-1)

    vals, idx = pl.pallas_call(
        _knn_body,
        grid=(b, n // _TI),
        in_specs=[pl.BlockSpec((1, _TI, d), lambda bi, ti: (bi, ti, 0)),
                  pl.BlockSpec((1, n, d), lambda bi, ti: (bi, 0, 0)),
                  pl.BlockSpec((1, _TI, 1), lambda bi, ti: (bi, ti, 0)),
                  pl.BlockSpec((1, 1, n), lambda bi, ti: (bi, 0, 0))],
        out_specs=[pl.BlockSpec((1, _TI, k), lambda bi, ti: (bi, ti, 0)),
                   pl.BlockSpec((1, _TI, k), lambda bi, ti: (bi, ti, 0))],
        out_shape=[jax.ShapeDtypeStruct((b, n, k), jnp.float32),
                   jax.ShapeDtypeStruct((b, n, k), jnp.int32)],
    )(z, z, sq.reshape(b, n, 1), sq.reshape(b, 1, n))

    t = jnp.exp(jnp.clip(temperature, -5.0, 5.0))
    logprobs = -vals * t
    idx_flat = idx.reshape(b, n * k)
    rows = jnp.broadcast_to(jnp.arange(n, dtype=idx.dtype)[None, :, None],
                            (b, n, k)).reshape(b, n * k)
    edges = jnp.stack((idx_flat, rows), axis=-1)
    return edges, logprobs


# speedup vs baseline: 1692.0019x; 1.1681x over previous
import jax
import jax.numpy as jnp
from jax import lax
from jax.experimental import pallas as pl

_K = 5
_BM = 2048
_TI = 512


def _mlp_body(x_ref, w1_ref, b1_ref, w2_ref, b2_ref, w3_ref, b3_ref, z_ref):
    h = jnp.dot(x_ref[...], w1_ref[...], preferred_element_type=jnp.float32)
    h = jnp.maximum(h + b1_ref[...], 0.0)
    h = jnp.dot(h, w2_ref[...], preferred_element_type=jnp.float32)
    h = jnp.maximum(h + b2_ref[...], 0.0)
    h = jnp.dot(h, w3_ref[...], preferred_element_type=jnp.float32)
    z_ref[...] = h + b3_ref[...]


def _knn_body(zq_ref, zk_ref, sqc_ref, sqr_ref, vals_ref, idx_ref):
    zq = zq_ref[0]
    zk = zk_ref[0]
    sqq = sqc_ref[0]
    sqk = sqr_ref[0]
    n = zk.shape[0]
    cross = lax.dot_general(zq, zk, (((1,), (1,)), ((), ())),
                            preferred_element_type=jnp.float32)
    dist = jnp.maximum(sqq + sqk - 2.0 * cross, 0.0)
    iota = lax.broadcasted_iota(jnp.int32, dist.shape, 1)
    vals, idxs = [], []
    cur = dist
    for _ in range(_K):
        m = jnp.min(cur, axis=1, keepdims=True)
        im = jnp.min(jnp.where(cur == m, iota, n), axis=1, keepdims=True)
        vals.append(m)
        idxs.append(im)
        cur = jnp.where(iota == im, jnp.inf, cur)
    vals_ref[0] = jnp.concatenate(vals, axis=1)
    idx_ref[0] = jnp.concatenate(idxs, axis=1)


def kernel(x, temperature, W1, b1, W2, b2, W3, b3):
    b, n, d = x.shape
    k = _K
    x2 = x.reshape(b * n, d)
    wspec = pl.BlockSpec((d, d), lambda i: (0, 0))
    bspec = pl.BlockSpec((1, d), lambda i: (0, 0))
    z2 = pl.pallas_call(
        _mlp_body,
        grid=(b * n // _BM,),
        in_specs=[pl.BlockSpec((_BM, d), lambda i: (i, 0)),
                  wspec, bspec, wspec, bspec, wspec, bspec],
        out_specs=pl.BlockSpec((_BM, d), lambda i: (i, 0)),
        out_shape=jax.ShapeDtypeStruct((b * n, d), jnp.float32),
    )(x2, W1.T, b1.reshape(1, d), W2.T, b2.reshape(1, d), W3.T,
      b3.reshape(1, d))
    z = z2.reshape(b, n, d)
    sq = jnp.sum(z * z, axis=-1)

    vals, idx = pl.pallas_call(
        _knn_body,
        grid=(b, n // _TI),
        in_specs=[pl.BlockSpec((1, _TI, d), lambda bi, ti: (bi, ti, 0)),
                  pl.BlockSpec((1, n, d), lambda bi, ti: (bi, 0, 0)),
                  pl.BlockSpec((1, _TI, 1), lambda bi, ti: (bi, ti, 0)),
                  pl.BlockSpec((1, 1, n), lambda bi, ti: (bi, 0, 0))],
        out_specs=[pl.BlockSpec((1, _TI, k), lambda bi, ti: (bi, ti, 0)),
                   pl.BlockSpec((1, _TI, k), lambda bi, ti: (bi, ti, 0))],
        out_shape=[jax.ShapeDtypeStruct((b, n, k), jnp.float32),
                   jax.ShapeDtypeStruct((b, n, k), jnp.int32)],
    )(z, z, sq.reshape(b, n, 1), sq.reshape(b, 1, n))

    t = jnp.exp(jnp.clip(temperature, -5.0, 5.0))
    logprobs = -vals * t
    idx_flat = idx.reshape(b, n * k)
    rows = jnp.broadcast_to(jnp.arange(n, dtype=idx.dtype)[None, :, None],
                            (b, n, k)).reshape(b, n * k)
    edges = jnp.stack((idx_flat, rows), axis=-1)
    return edges, logprobs


# speedup vs baseline: 1715.2919x; 1.0138x over previous
import jax
import jax.numpy as jnp
from jax import lax
from jax.experimental import pallas as pl

_K = 5
_BM = 2048
_TI = 1024


def _mlp_body(x_ref, w1_ref, b1_ref, w2_ref, b2_ref, w3_ref, b3_ref, z_ref):
    h = jnp.dot(x_ref[...], w1_ref[...], preferred_element_type=jnp.float32)
    h = jnp.maximum(h + b1_ref[...], 0.0)
    h = jnp.dot(h, w2_ref[...], preferred_element_type=jnp.float32)
    h = jnp.maximum(h + b2_ref[...], 0.0)
    h = jnp.dot(h, w3_ref[...], preferred_element_type=jnp.float32)
    z_ref[...] = h + b3_ref[...]


def _knn_body(zq_ref, zk_ref, sqc_ref, sqr_ref, vals_ref, idx_ref):
    zq = zq_ref[0]
    zk = zk_ref[0]
    sqq = sqc_ref[0]
    sqk = sqr_ref[0]
    n = zk.shape[0]
    cross = lax.dot_general(zq, zk, (((1,), (1,)), ((), ())),
                            preferred_element_type=jnp.float32)
    dist = jnp.maximum(sqq + sqk - 2.0 * cross, 0.0)
    iota = lax.broadcasted_iota(jnp.int32, dist.shape, 1)
    vals, idxs = [], []
    cur = dist
    for _ in range(_K):
        m = jnp.min(cur, axis=1, keepdims=True)
        im = jnp.min(jnp.where(cur == m, iota, n), axis=1, keepdims=True)
        vals.append(m)
        idxs.append(im)
        cur = jnp.where(iota == im, jnp.inf, cur)
    vals_ref[0] = jnp.concatenate(vals, axis=1)
    idx_ref[0] = jnp.concatenate(idxs, axis=1)


def kernel(x, temperature, W1, b1, W2, b2, W3, b3):
    b, n, d = x.shape
    k = _K
    x2 = x.reshape(b * n, d)
    wspec = pl.BlockSpec((d, d), lambda i: (0, 0))
    bspec = pl.BlockSpec((1, d), lambda i: (0, 0))
    z2 = pl.pallas_call(
        _mlp_body,
        grid=(b * n // _BM,),
        in_specs=[pl.BlockSpec((_BM, d), lambda i: (i, 0)),
                  wspec, bspec, wspec, bspec, wspec, bspec],
        out_specs=pl.BlockSpec((_BM, d), lambda i: (i, 0)),
        out_shape=jax.ShapeDtypeStruct((b * n, d), jnp.float32),
    )(x2, W1.T, b1.reshape(1, d), W2.T, b2.reshape(1, d), W3.T,
      b3.reshape(1, d))
    z = z2.reshape(b, n, d)
    sq = jnp.sum(z * z, axis=-1)

    vals, idx = pl.pallas_call(
        _knn_body,
        grid=(b, n // _TI),
        in_specs=[pl.BlockSpec((1, _TI, d), lambda bi, ti: (bi, ti, 0)),
                  pl.BlockSpec((1, n, d), lambda bi, ti: (bi, 0, 0)),
                  pl.BlockSpec((1, _TI, 1), lambda bi, ti: (bi, ti, 0)),
                  pl.BlockSpec((1, 1, n), lambda bi, ti: (bi, 0, 0))],
        out_specs=[pl.BlockSpec((1, _TI, k), lambda bi, ti: (bi, ti, 0)),
                   pl.BlockSpec((1, _TI, k), lambda bi, ti: (bi, ti, 0))],
        out_shape=[jax.ShapeDtypeStruct((b, n, k), jnp.float32),
                   jax.ShapeDtypeStruct((b, n, k), jnp.int32)],
    )(z, z, sq.reshape(b, n, 1), sq.reshape(b, 1, n))

    t = jnp.exp(jnp.clip(temperature, -5.0, 5.0))
    logprobs = -vals * t
    idx_flat = idx.reshape(b, n * k)
    rows = jnp.broadcast_to(jnp.arange(n, dtype=idx.dtype)[None, :, None],
                            (b, n, k)).reshape(b, n * k)
    edges = jnp.stack((idx_flat, rows), axis=-1)
    return edges, logprobs


# speedup vs baseline: 1935.3595x; 1.1283x over previous
import jax
import jax.numpy as jnp
from jax import lax
from jax.experimental import pallas as pl

_K = 5
_BM = 2048
_TI = 1024


def _mlp_body(x_ref, w1_ref, b1_ref, w2_ref, b2_ref, w3_ref, b3_ref, z_ref):
    h = jnp.dot(x_ref[...], w1_ref[...], preferred_element_type=jnp.float32)
    h = jnp.maximum(h + b1_ref[...], 0.0)
    h = jnp.dot(h, w2_ref[...], preferred_element_type=jnp.float32)
    h = jnp.maximum(h + b2_ref[...], 0.0)
    h = jnp.dot(h, w3_ref[...], preferred_element_type=jnp.float32)
    z_ref[...] = h + b3_ref[...]


def _knn_body(zq_ref, zk_ref, sqc_ref, sqr_ref, vals_ref, idx_ref):
    zq = zq_ref[0]
    zk = zk_ref[0]
    sqq = sqc_ref[0]
    sqk = sqr_ref[0]
    n = zk.shape[0]
    cross = lax.dot_general(zq, zk, (((1,), (1,)), ((), ())),
                            preferred_element_type=jnp.float32)
    dist = jnp.maximum(sqq + sqk - 2.0 * cross, 0.0)
    iota = lax.broadcasted_iota(jnp.int32, dist.shape, 1).astype(jnp.float32)
    nf = jnp.float32(n)
    vals, idxs = [], []
    cur = dist
    for _ in range(_K):
        m = jnp.min(cur, axis=1, keepdims=True)
        im = jnp.min(jnp.where(cur == m, iota, nf), axis=1, keepdims=True)
        vals.append(m)
        idxs.append(im)
        cur = jnp.where(iota == im, jnp.inf, cur)
    vals_ref[0] = jnp.concatenate(vals, axis=1)
    idx_ref[0] = jnp.concatenate(idxs, axis=1).astype(jnp.int32)


def kernel(x, temperature, W1, b1, W2, b2, W3, b3):
    b, n, d = x.shape
    k = _K
    x2 = x.reshape(b * n, d)
    wspec = pl.BlockSpec((d, d), lambda i: (0, 0))
    bspec = pl.BlockSpec((1, d), lambda i: (0, 0))
    z2 = pl.pallas_call(
        _mlp_body,
        grid=(b * n // _BM,),
        in_specs=[pl.BlockSpec((_BM, d), lambda i: (i, 0)),
                  wspec, bspec, wspec, bspec, wspec, bspec],
        out_specs=pl.BlockSpec((_BM, d), lambda i: (i, 0)),
        out_shape=jax.ShapeDtypeStruct((b * n, d), jnp.float32),
    )(x2, W1.T, b1.reshape(1, d), W2.T, b2.reshape(1, d), W3.T,
      b3.reshape(1, d))
    z = z2.reshape(b, n, d)
    sq = jnp.sum(z * z, axis=-1)

    vals, idx = pl.pallas_call(
        _knn_body,
        grid=(b, n // _TI),
        in_specs=[pl.BlockSpec((1, _TI, d), lambda bi, ti: (bi, ti, 0)),
                  pl.BlockSpec((1, n, d), lambda bi, ti: (bi, 0, 0)),
                  pl.BlockSpec((1, _TI, 1), lambda bi, ti: (bi, ti, 0)),
                  pl.BlockSpec((1, 1, n), lambda bi, ti: (bi, 0, 0))],
        out_specs=[pl.BlockSpec((1, _TI, k), lambda bi, ti: (bi, ti, 0)),
                   pl.BlockSpec((1, _TI, k), lambda bi, ti: (bi, ti, 0))],
        out_shape=[jax.ShapeDtypeStruct((b, n, k), jnp.float32),
                   jax.ShapeDtypeStruct((b, n, k), jnp.int32)],
    )(z, z, sq.reshape(b, n, 1), sq.reshape(b, 1, n))

    t = jnp.exp(jnp.clip(temperature, -5.0, 5.0))
    logprobs = -vals * t
    idx_flat = idx.reshape(b, n * k)
    rows = jnp.broadcast_to(jnp.arange(n, dtype=idx.dtype)[None, :, None],
                            (b, n, k)).reshape(b, n * k)
    edges = jnp.stack((idx_flat, rows), axis=-1)
    return edges, logprobs
